# baseline (speedup 1.0000x reference)
import functools

import jax
import jax.numpy as jnp
from jax import lax
from jax.experimental import pallas as pl
from jax.experimental.pallas import tpu as pltpu
from jax.experimental.pallas import tpu_sc as plsc

K = 8
NSEL = 12
NSLOT = 16
CH = 128
BIG = 3.0e38
IBIG = 2**30


def _k1_body(x_ref, xt_ref, mask_ref, st_ref, mg_ref):
    qb = x_ref.shape[0]
    nb_cols = xt_ref.shape[1]
    x = x_ref[...]
    xt = xt_ref[...]
    dot = lax.dot_general(
        x, xt, (((1,), (0,)), ((), ())),
        preferred_element_type=jnp.float32,
        precision=lax.Precision.DEFAULT,
    )
    t2 = jnp.sum(xt * xt, axis=0)
    x2 = jnp.sum(x * x, axis=1)
    s = (x2[:, None] + t2[None, :]) - 2.0 * dot
    s = jnp.maximum(s, 0.0) + mask_ref[...][None, :]
    st_ref[...] = s
    m = jnp.min(s.reshape(qb, nb_cols // CH, CH), axis=2)
    mg_ref[...] = m.T


def _k2_body(n_groups, dummy_g, mg_ref, out_ref):
    i = pl.program_id(0)
    qb = mg_ref.shape[1]
    cand = mg_ref[...]
    row = jax.lax.broadcasted_iota(jnp.int32, (n_groups, qb), 0)
    gids = []
    for _ in range(NSEL):
        a = jnp.argmin(cand, axis=0).astype(jnp.int32)
        gids.append(a)
        cand = jnp.where(row == a[None, :], BIG, cand)
    for _ in range(NSLOT - NSEL):
        gids.append(jnp.full((qb,), dummy_g, jnp.int32))
    g = jnp.stack(gids, axis=0)
    qidx = jax.lax.broadcasted_iota(jnp.int32, (NSLOT, qb), 1) + i * qb
    out_ref[...] = qidx * n_groups + g


def _sc_gather(table, idx):
    info = plsc.get_sparse_core_info()
    nw = info.num_cores * info.num_subcores
    nslot, qdim = idx.shape
    b = nslot * qdim
    rounds = 4
    part = b // nw // rounds
    w_per_slot = qdim // (part * rounds)
    d = table.shape[1]
    mesh = plsc.VectorSubcoreMesh(core_axis_name="c", subcore_axis_name="s")

    @functools.partial(
        pl.kernel, mesh=mesh,
        out_type=jax.ShapeDtypeStruct((b, d), jnp.float32),
        scratch_types=[
            pltpu.VMEM((part,), jnp.int32),
            pltpu.VMEM((part, d), jnp.float32),
            pltpu.SemaphoreType.DMA,
        ],
    )
    def k(table_hbm, idx_hbm, out_hbm, idx_v, rows_v, sem):
        wid = lax.axis_index("s") * info.num_cores + lax.axis_index("c")
        slot = wid // w_per_slot
        for h in range(rounds):
            qbase = (wid % w_per_slot) * (part * rounds) + h * part
            pltpu.sync_copy(idx_hbm.at[slot, pl.ds(qbase, part)], idx_v)
            pltpu.async_copy(table_hbm.at[idx_v], rows_v, sem).wait()
            pltpu.sync_copy(rows_v, out_hbm.at[pl.ds(slot * qdim + qbase, part)])

    return k(table, idx)


def _k4_body(n_groups, sv_ref, g_ref, y2_ref, out_ref):
    qb = sv_ref.shape[1]
    sv = sv_ref[...]
    g = jax.lax.rem(g_ref[...], n_groups)
    lanes = jax.lax.broadcasted_iota(jnp.int32, (NSLOT, qb, CH), 2)
    orig = g[:, :, None] * CH + lanes

    top_i = []
    for _ in range(K):
        vm = jnp.min(jnp.min(sv, axis=0), axis=1)
        hit = sv == vm[None, :, None]
        li = jnp.min(jnp.min(jnp.where(hit, orig, IBIG), axis=0), axis=1)
        top_i.append(li)
        sv = jnp.where(hit & (orig == li[None, :, None]), BIG, sv)

    y2 = y2_ref[...]
    r_dim = y2.shape[0]
    labels = []
    for gi in top_i:
        r = gi // 128
        c = gi - r * 128
        oh_r = (jax.lax.broadcasted_iota(jnp.int32, (qb, r_dim), 1)
                == r[:, None]).astype(jnp.float32)
        rowv = jax.lax.dot_general(
            oh_r, y2, (((1,), (0,)), ((), ())),
            preferred_element_type=jnp.float32,
        )
        oh_c = (jax.lax.broadcasted_iota(jnp.int32, (qb, 128), 1)
                == c[:, None]).astype(jnp.float32)
        labels.append(jnp.sum(rowv * oh_c, axis=1))

    counts = []
    for k in range(K):
        cnt = jnp.zeros((qb,), jnp.float32)
        for m in range(K):
            cnt = cnt + (labels[k] == labels[m]).astype(jnp.float32)
        counts.append(cnt)
    keys = [counts[k] * 1024.0 - labels[k] for k in range(K)]
    best = keys[0]
    for k in range(1, K):
        best = jnp.maximum(best, keys[k])
    y = jnp.full((qb,), 1.0e9, jnp.float32)
    for k in range(K):
        y = jnp.minimum(y, jnp.where(keys[k] == best, labels[k], 1.0e9))
    out_ref[...] = y.astype(jnp.int32)


def kernel(X, X_train, y_train):
    q, d = X.shape
    n = X_train.shape[0]
    qb = 256
    nb_cols = 4096
    n_qb = q // qb
    nb = -(-n // nb_cols)
    n_pad = nb * nb_cols
    n_groups = n_pad // CH

    xt = jnp.pad(X_train.T, ((0, 0), (0, n_pad - n)))
    mask = jnp.where(jnp.arange(n_pad) < n, 0.0, BIG).astype(jnp.float32)
    y2d = (jnp.pad(y_train.astype(jnp.float32), (0, n_pad - n))
           .reshape(n_groups, CH))

    st, mg = pl.pallas_call(
        _k1_body,
        grid=(nb, n_qb),
        in_specs=[
            pl.BlockSpec((qb, d), lambda j, i: (i, 0)),
            pl.BlockSpec((d, nb_cols), lambda j, i: (0, j)),
            pl.BlockSpec((nb_cols,), lambda j, i: (j,)),
        ],
        out_specs=[
            pl.BlockSpec((qb, nb_cols), lambda j, i: (i, j)),
            pl.BlockSpec((nb_cols // CH, qb), lambda j, i: (j, i)),
        ],
        out_shape=[
            jax.ShapeDtypeStruct((q, n_pad), jnp.float32),
            jax.ShapeDtypeStruct((n_groups, q), jnp.float32),
        ],
    )(X, xt, mask)

    flat = pl.pallas_call(
        functools.partial(_k2_body, n_groups, n_groups - 1),
        grid=(n_qb,),
        in_specs=[pl.BlockSpec((n_groups, qb), lambda i: (0, i))],
        out_specs=pl.BlockSpec((NSLOT, qb), lambda i: (0, i)),
        out_shape=jax.ShapeDtypeStruct((NSLOT, q), jnp.int32),
    )(mg)

    sv = _sc_gather(st.reshape(q * n_groups, CH)[:8192],
                    jax.lax.rem(flat, 8192))
    return (sv[:, 0], mg[0])

    out = pl.pallas_call(
        functools.partial(_k4_body, n_groups),
        grid=(n_qb,),
        in_specs=[
            pl.BlockSpec((NSLOT, qb, CH), lambda i: (0, i, 0)),
            pl.BlockSpec((NSLOT, qb), lambda i: (0, i)),
            pl.BlockSpec((n_groups, CH), lambda i: (0, 0)),
        ],
        out_specs=pl.BlockSpec((qb,), lambda i: (i,)),
        out_shape=jax.ShapeDtypeStruct((q,), jnp.int32),
    )(sv.reshape(NSLOT, q, CH), flat, y2d)
    return out


# speedup vs baseline: 1.8933x; 1.8933x over previous
import functools

import jax
import jax.numpy as jnp
from jax import lax
from jax.experimental import pallas as pl
from jax.experimental.pallas import tpu as pltpu
from jax.experimental.pallas import tpu_sc as plsc

K = 8
NSEL = 12
NSLOT = 16
CH = 128
BIG = 3.0e38
IBIG = 2**30


def _k1_body(x_ref, xt_ref, mask_ref, st_ref, mg_ref):
    qb = x_ref.shape[0]
    nb_cols = xt_ref.shape[1]
    x = x_ref[...]
    xt = xt_ref[...]
    dot = lax.dot_general(
        x, xt, (((1,), (0,)), ((), ())),
        preferred_element_type=jnp.float32,
        precision=lax.Precision.DEFAULT,
    )
    t2 = jnp.sum(xt * xt, axis=0)
    x2 = jnp.sum(x * x, axis=1)
    s = (x2[:, None] + t2[None, :]) - 2.0 * dot
    s = jnp.maximum(s, 0.0) + mask_ref[...][None, :]
    st_ref[...] = jnp.stack(
        [s[:, v * CH:(v + 1) * CH] for v in range(nb_cols // CH)], axis=0)
    m = jnp.min(s.reshape(qb, nb_cols // CH, CH), axis=2)
    mg_ref[...] = m.T


def _k2_body(q_total, dummy_g, mg_ref, out_ref):
    i = pl.program_id(0)
    qb = mg_ref.shape[1]
    cand = mg_ref[...]
    n_groups = cand.shape[0]
    row = jax.lax.broadcasted_iota(jnp.int32, (n_groups, qb), 0)
    gids = []
    for _ in range(NSEL):
        a = jnp.argmin(cand, axis=0).astype(jnp.int32)
        gids.append(a)
        cand = jnp.where(row == a[None, :], BIG, cand)
    for _ in range(NSLOT - NSEL):
        gids.append(jnp.full((qb,), dummy_g, jnp.int32))
    g = jnp.stack(gids, axis=0)
    qidx = jax.lax.broadcasted_iota(jnp.int32, (NSLOT, qb), 1) + i * qb
    out_ref[...] = g * q_total + qidx


def _sc_gather(table, idx):
    info = plsc.get_sparse_core_info()
    nw = info.num_cores * info.num_subcores
    nslot, qdim = idx.shape
    b = nslot * qdim
    rounds = 4
    part = b // nw // rounds
    w_per_slot = qdim // (part * rounds)
    d = table.shape[1]
    mesh = plsc.VectorSubcoreMesh(core_axis_name="c", subcore_axis_name="s")

    @functools.partial(
        pl.kernel, mesh=mesh,
        out_type=jax.ShapeDtypeStruct((b, d), jnp.float32),
        scratch_types=[
            pltpu.VMEM((part,), jnp.int32),
            pltpu.VMEM((part, d), jnp.float32),
            pltpu.SemaphoreType.DMA,
        ],
    )
    def k(table_hbm, idx_hbm, out_hbm, idx_v, rows_v, sem):
        wid = lax.axis_index("s") * info.num_cores + lax.axis_index("c")
        slot = wid // w_per_slot
        for h in range(rounds):
            qbase = (wid % w_per_slot) * (part * rounds) + h * part
            pltpu.sync_copy(idx_hbm.at[slot, pl.ds(qbase, part)], idx_v)
            pltpu.async_copy(table_hbm.at[idx_v], rows_v, sem).wait()
            pltpu.sync_copy(rows_v, out_hbm.at[pl.ds(slot * qdim + qbase, part)])

    return k(table, idx)


def _k4_body(q_total, sv_ref, g_ref, y2_ref, out_ref):
    qb = sv_ref.shape[1]
    sv = sv_ref[...]
    g = g_ref[...] // q_total
    lanes = jax.lax.broadcasted_iota(jnp.int32, (NSLOT, qb, CH), 2)
    orig = g[:, :, None] * CH + lanes

    top_i = []
    for _ in range(K):
        vm = jnp.min(jnp.min(sv, axis=0), axis=1)
        hit = sv == vm[None, :, None]
        li = jnp.min(jnp.min(jnp.where(hit, orig, IBIG), axis=0), axis=1)
        top_i.append(li)
        sv = jnp.where(hit & (orig == li[None, :, None]), BIG, sv)

    y2 = y2_ref[...]
    r_dim = y2.shape[0]
    labels = []
    for gi in top_i:
        r = gi // 128
        c = gi - r * 128
        oh_r = (jax.lax.broadcasted_iota(jnp.int32, (qb, r_dim), 1)
                == r[:, None]).astype(jnp.float32)
        rowv = jax.lax.dot_general(
            oh_r, y2, (((1,), (0,)), ((), ())),
            preferred_element_type=jnp.float32,
        )
        oh_c = (jax.lax.broadcasted_iota(jnp.int32, (qb, 128), 1)
                == c[:, None]).astype(jnp.float32)
        labels.append(jnp.sum(rowv * oh_c, axis=1))

    counts = []
    for k in range(K):
        cnt = jnp.zeros((qb,), jnp.float32)
        for m in range(K):
            cnt = cnt + (labels[k] == labels[m]).astype(jnp.float32)
        counts.append(cnt)
    keys = [counts[k] * 1024.0 - labels[k] for k in range(K)]
    best = keys[0]
    for k in range(1, K):
        best = jnp.maximum(best, keys[k])
    y = jnp.full((qb,), 1.0e9, jnp.float32)
    for k in range(K):
        y = jnp.minimum(y, jnp.where(keys[k] == best, labels[k], 1.0e9))
    out_ref[...] = y.astype(jnp.int32)


def kernel(X, X_train, y_train):
    q, d = X.shape
    n = X_train.shape[0]
    qb = 256
    nb_cols = 4096
    n_qb = q // qb
    nb = -(-n // nb_cols)
    n_pad = nb * nb_cols
    n_groups = n_pad // CH

    xt = jnp.pad(X_train.T, ((0, 0), (0, n_pad - n)))
    mask = jnp.where(jnp.arange(n_pad) < n, 0.0, BIG).astype(jnp.float32)
    y2d = (jnp.pad(y_train.astype(jnp.float32), (0, n_pad - n))
           .reshape(n_groups, CH))

    st, mg = pl.pallas_call(
        _k1_body,
        grid=(nb, n_qb),
        in_specs=[
            pl.BlockSpec((qb, d), lambda j, i: (i, 0)),
            pl.BlockSpec((d, nb_cols), lambda j, i: (0, j)),
            pl.BlockSpec((nb_cols,), lambda j, i: (j,)),
        ],
        out_specs=[
            pl.BlockSpec((nb_cols // CH, qb, CH), lambda j, i: (j, i, 0)),
            pl.BlockSpec((nb_cols // CH, qb), lambda j, i: (j, i)),
        ],
        out_shape=[
            jax.ShapeDtypeStruct((n_groups, q, CH), jnp.float32),
            jax.ShapeDtypeStruct((n_groups, q), jnp.float32),
        ],
    )(X, xt, mask)

    flat = pl.pallas_call(
        functools.partial(_k2_body, q, n_groups - 1),
        grid=(n_qb,),
        in_specs=[pl.BlockSpec((n_groups, qb), lambda i: (0, i))],
        out_specs=pl.BlockSpec((NSLOT, qb), lambda i: (0, i)),
        out_shape=jax.ShapeDtypeStruct((NSLOT, q), jnp.int32),
    )(mg)

    sv = _sc_gather(st.reshape(q * n_groups, CH), flat)

    out = pl.pallas_call(
        functools.partial(_k4_body, q),
        grid=(n_qb,),
        in_specs=[
            pl.BlockSpec((NSLOT, qb, CH), lambda i: (0, i, 0)),
            pl.BlockSpec((NSLOT, qb), lambda i: (0, i)),
            pl.BlockSpec((n_groups, CH), lambda i: (0, 0)),
        ],
        out_specs=pl.BlockSpec((qb,), lambda i: (i,)),
        out_shape=jax.ShapeDtypeStruct((q,), jnp.int32),
    )(sv.reshape(NSLOT, q, CH), flat, y2d)
    return out
